# baseline (speedup 1.0000x reference)
import functools

import jax
import jax.numpy as jnp
import numpy as np
from jax import lax
from jax.experimental import pallas as pl
from jax.experimental.pallas import tpu as pltpu
from jax.experimental.pallas import tpu_sc as plsc

N_PROP = 5000
N_CLS = 8
N_CAND = N_PROP * N_CLS
NT = 16
CHUNK = 2560
ROWS_PER_TILE = CHUNK // N_CLS
NV = CHUNK // 16
SCORE_THR = 0.95
IOU_THR = 0.5
MAX_NUM = 100
KEEP_PAD = 112
N_ATTR = 16
NREC = 8
BIG = np.int32(np.iinfo(np.int32).max)
NEG_INF = np.float32(-np.inf)
F0 = np.float32(0.0)


def _sc_body(scores_hbm, bb_hbm, attrs_hbm, mn_hbm,
             out1_hbm, out2_hbm,
             sch9, bbch, scorec, candrec,
             recv, allv, keepi, keeps, rowidx, attrrows, outv, mnv,
             bufA, winners, sem):
    sid = lax.axis_index("s")
    base = sid * CHUNK
    lane = jnp.arange(16, dtype=jnp.int32)
    zeros16i = jnp.zeros((16,), jnp.int32)
    ones16i = jnp.full((16,), 1, jnp.int32)

    pltpu.sync_copy(
        scores_hbm.at[pl.ds(sid * ROWS_PER_TILE * 9, ROWS_PER_TILE * 9)], sch9)
    pltpu.sync_copy(
        bb_hbm.at[pl.ds(sid * ROWS_PER_TILE * 4, ROWS_PER_TILE * 4)], bbch)
    pltpu.sync_copy(mn_hbm, mnv)

    def initb(k, _):
        scorec[pl.ds(k * 16, 16)] = jnp.full((16,), NEG_INF, jnp.float32)
        return 0
    lax.fori_loop(0, NV, initb, 0)

    def scan_body(k, carry):
        off, mc, ffl = carry
        j = k * 16 + lane
        rl = j >> 3
        cls = j & 7
        v = plsc.load_gather(sch9, [rl * 9 + 1 + cls])
        msk = v > SCORE_THR
        fidx_v = base + j
        cum = plsc.cumsum(msk.astype(jnp.int32))
        pos = off + cum - 1
        plsc.store_scatter(candrec, [pos * NREC],
                           plsc.bitcast(fidx_v, jnp.float32), mask=msk)
        plsc.store_scatter(scorec, [pos], v, mask=msk)
        cnt = plsc.all_reduce_population_count(msk)
        bx1 = plsc.load_gather(bbch, [rl * 4])
        by1 = plsc.load_gather(bbch, [rl * 4 + 1])
        bx2 = plsc.load_gather(bbch, [rl * 4 + 2])
        by2 = plsc.load_gather(bbch, [rl * 4 + 3])
        cmax = jnp.maximum(jnp.maximum(bx1, by1), jnp.maximum(bx2, by2))
        mc = jnp.maximum(mc, jnp.where(msk, cmax, NEG_INF))
        ffl = jnp.minimum(ffl, jnp.where(msk, fidx_v, BIG))
        return (off + cnt, mc, ffl)

    off_v, mc_v, ffl_v = lax.fori_loop(
        0, NV, scan_body,
        (zeros16i, jnp.full((16,), NEG_INF, jnp.float32),
         jnp.full((16,), BIG, jnp.int32)))
    nloc = jnp.max(off_v)
    mcl = jnp.max(mc_v)
    ffl = jnp.min(ffl_v)
    fsl = jnp.max(plsc.load_gather(scorec, [zeros16i]))
    fsl = jnp.where(nloc > 0, fsl, F0)
    s0 = jnp.max(plsc.load_gather(sch9, [ones16i]))

    rec = jnp.full((16,), F0, jnp.float32)
    rec = jnp.where(lane == 0, mcl, rec)
    rec = jnp.where(lane == 1,
                    plsc.bitcast(jnp.full((16,), ffl, jnp.int32), jnp.float32),
                    rec)
    rec = jnp.where(lane == 2, fsl, rec)
    rec = jnp.where(lane == 3, s0, rec)
    recv[...] = rec
    pltpu.sync_copy(recv, bufA.at[pl.ds(sid * 16, 16)])
    plsc.subcore_barrier()
    pltpu.sync_copy(bufA, allv)

    def field_f(f):
        return plsc.load_gather(allv, [lane * 16 + f])

    mc_g = jnp.max(field_f(0))
    ffl_all = plsc.bitcast(field_f(1), jnp.int32)
    gfill = jnp.min(ffl_all)
    gfs = jnp.max(jnp.where(ffl_all == gfill, field_f(2), NEG_INF))
    s0g = jnp.max(jnp.where(lane == 0, field_f(3), NEG_INF))
    none_valid = gfill == BIG
    gf = jnp.where(none_valid, 0, gfill)
    gfs = jnp.where(none_valid, s0g, gfs)
    offc = mc_g + np.float32(1.0)

    nvl = (nloc + 15) // 16

    def gco(k, _):
        cid = k * 16 + lane
        idxv = plsc.bitcast(plsc.load_gather(candrec, [cid * NREC]), jnp.int32)
        lidx = jnp.minimum(jnp.maximum(idxv - base, 0), CHUNK - 1)
        rl = lidx >> 3
        labf = (idxv & 7).astype(jnp.float32)
        offv = labf * offc
        a = plsc.load_gather(bbch, [rl * 4]) + offv
        b = plsc.load_gather(bbch, [rl * 4 + 1]) + offv
        c = plsc.load_gather(bbch, [rl * 4 + 2]) + offv
        d = plsc.load_gather(bbch, [rl * 4 + 3]) + offv
        area = jnp.maximum(c - a, F0) * jnp.maximum(d - b, F0)
        plsc.store_scatter(candrec, [cid * NREC + 1], a)
        plsc.store_scatter(candrec, [cid * NREC + 2], b)
        plsc.store_scatter(candrec, [cid * NREC + 3], c)
        plsc.store_scatter(candrec, [cid * NREC + 4], d)
        plsc.store_scatter(candrec, [cid * NREC + 5], area)
        return 0
    lax.fori_loop(0, nvl, gco, 0)

    def pfill(k, _):
        keepi[pl.ds(k * 16, 16)] = jnp.full((16,), gf, jnp.int32)
        keeps[pl.ds(k * 16, 16)] = jnp.full((16,), gfs, jnp.float32)
        return 0
    lax.fori_loop(0, KEEP_PAD, pfill, 0)

    mn_s = jnp.max(mnv[...])
    limit = jnp.minimum(jnp.int32(MAX_NUM), mn_s)

    recmap = jnp.minimum(jnp.maximum(lane - 1, 0), NREC - 1)

    def cond(carry):
        t, alive = carry
        return jnp.logical_and(t < limit, alive == 1)

    def body(carry):
        t, _ = carry

        def am(k, c2):
            best, bk = c2
            v = scorec[pl.ds(k * 16, 16)]
            upd = v > best
            return (jnp.where(upd, v, best),
                    jnp.where(upd, jnp.full((16,), k, jnp.int32), bk))
        best, bk = lax.fori_loop(
            0, nvl, am,
            (jnp.full((16,), NEG_INF, jnp.float32), zeros16i))
        m_l = jnp.max(best)
        pos_pl = bk * 16 + lane
        pfv = plsc.bitcast(plsc.load_gather(candrec, [pos_pl * NREC]),
                           jnp.int32)
        tied = best == m_l
        jl = jnp.min(jnp.where(tied, pfv, BIG))
        posl = jnp.min(jnp.where(jnp.logical_and(tied, pfv == jl), pos_pl, BIG))
        posc = jnp.minimum(jnp.maximum(posl, 0), CHUNK - 1)
        r = plsc.load_gather(
            candrec, [jnp.full((16,), posc * NREC, jnp.int32) + recmap])
        r = jnp.where(lane == 0, m_l, r)
        recv[...] = r
        tb = jnp.bitwise_and(t, 1)
        pltpu.sync_copy(recv, winners.at[pl.ds(tb * 256 + sid * 16, 16)])
        plsc.subcore_barrier()
        pltpu.sync_copy(winners.at[pl.ds(tb * 256, 256)], allv)

        sco = field_f(0)
        idxf = plsc.bitcast(field_f(1), jnp.int32)
        m_g = jnp.max(sco)
        alive = m_g > NEG_INF
        gtied = sco == m_g
        jwin = jnp.min(jnp.where(gtied, idxf, BIG))
        sel = jnp.logical_and(gtied, idxf == jwin)
        w16 = jnp.minimum(jnp.min(jnp.where(sel, lane, 16)), 15) * 16

        def wfield(f):
            return plsc.load_gather(allv, [jnp.full((16,), w16 + f, jnp.int32)])

        bx1 = wfield(2)
        by1 = wfield(3)
        bx2 = wfield(4)
        by2 = wfield(5)
        bar = wfield(6)

        @pl.when(alive)
        def _():
            keepi[pl.ds(t * 16, 16)] = jnp.full((16,), jwin, jnp.int32)
            keeps[pl.ds(t * 16, 16)] = jnp.full((16,), m_g, jnp.float32)

            def sup(k, _):
                c8 = (k * 16 + lane) * NREC
                fi = plsc.bitcast(plsc.load_gather(candrec, [c8]), jnp.int32)
                a1 = plsc.load_gather(candrec, [c8 + 1])
                b1 = plsc.load_gather(candrec, [c8 + 2])
                a2 = plsc.load_gather(candrec, [c8 + 3])
                b2 = plsc.load_gather(candrec, [c8 + 4])
                ar = plsc.load_gather(candrec, [c8 + 5])
                sc = scorec[pl.ds(k * 16, 16)]
                xx1 = jnp.maximum(bx1, a1)
                yy1 = jnp.maximum(by1, b1)
                xx2 = jnp.minimum(bx2, a2)
                yy2 = jnp.minimum(by2, b2)
                inter = (jnp.maximum(xx2 - xx1, F0) *
                         jnp.maximum(yy2 - yy1, F0))
                union = bar + ar - inter
                iou = inter / jnp.maximum(union, np.float32(1e-6))
                kill = jnp.logical_or(iou > IOU_THR, fi == jwin)
                scorec[pl.ds(k * 16, 16)] = jnp.where(kill, NEG_INF, sc)
                return 0
            lax.fori_loop(0, nvl, sup, 0)

        return (t + 1, alive.astype(jnp.int32))

    lax.while_loop(cond, body, (jnp.int32(0), jnp.int32(1)))

    @pl.when(sid == 0)
    def _():
        def ri(k, _):
            slots = k * 16 + lane
            kvv = plsc.load_gather(keepi, [slots * 16])
            rowidx[pl.ds(k * 16, 16)] = kvv >> 3
            return 0
        lax.fori_loop(0, KEEP_PAD // 16, ri, 0)
        pltpu.async_copy(attrs_hbm.at[rowidx], attrrows, sem).wait()

        def emit(tt, _):
            kv = keepi[pl.ds(tt * 16, 16)]
            sv = keeps[pl.ds(tt * 16, 16)]
            labf = (kv & 7).astype(jnp.float32)
            row = jnp.where(lane == 4, sv,
                            jnp.where(lane == 5, labf,
                                      jnp.zeros((16,), jnp.float32)))
            outv[pl.ds(tt * 16, 16)] = row
            return 0
        lax.fori_loop(0, KEEP_PAD, emit, 0)
        pltpu.sync_copy(outv, out1_hbm)
        pltpu.sync_copy(attrrows, out2_hbm)


_mesh = plsc.VectorSubcoreMesh(core_axis_name="c", subcore_axis_name="s")

_sc_call = functools.partial(
    pl.kernel,
    mesh=_mesh,
    out_type=[
        jax.ShapeDtypeStruct((KEEP_PAD * 16,), jnp.float32),
        jax.ShapeDtypeStruct((KEEP_PAD, N_ATTR), jnp.float32),
    ],
    compiler_params=pltpu.CompilerParams(needs_layout_passes=False,
                                         use_tc_tiling_on_sc=False),
    scratch_types=[
        pltpu.VMEM((ROWS_PER_TILE * 9,), jnp.float32),
        pltpu.VMEM((ROWS_PER_TILE * 4,), jnp.float32),
        pltpu.VMEM((CHUNK,), jnp.float32),
        pltpu.VMEM((CHUNK * NREC,), jnp.float32),
        pltpu.VMEM((16,), jnp.float32),
        pltpu.VMEM((256,), jnp.float32),
        pltpu.VMEM((KEEP_PAD * 16,), jnp.int32),
        pltpu.VMEM((KEEP_PAD * 16,), jnp.float32),
        pltpu.VMEM((KEEP_PAD,), jnp.int32),
        pltpu.VMEM((KEEP_PAD, N_ATTR), jnp.float32),
        pltpu.VMEM((KEEP_PAD * 16,), jnp.float32),
        pltpu.VMEM((16,), jnp.int32),
        pltpu.VMEM_SHARED((256,), jnp.float32),
        pltpu.VMEM_SHARED((512,), jnp.float32),
        pltpu.SemaphoreType.DMA,
    ],
)(_sc_body)


@jax.jit
def kernel(multi_bboxes, multi_scores, depth_pred, dim_pred, rot_pred,
           cen_2d_pred, max_num):
    attrs = jnp.concatenate([
        multi_bboxes,
        jnp.zeros((N_PROP, 2), jnp.float32),
        depth_pred,
        dim_pred,
        rot_pred,
        cen_2d_pred,
        jnp.zeros((N_PROP, 3), jnp.float32),
    ], axis=1)

    mn = jnp.full((16,), jnp.asarray(max_num, jnp.int32))

    n_pad_rows = NT * ROWS_PER_TILE - N_PROP
    s_flat = jnp.pad(multi_scores.reshape(-1), (0, n_pad_rows * 9))
    bb_flat = jnp.pad(multi_bboxes.reshape(-1), (0, n_pad_rows * 4))
    out1, out2 = _sc_call(s_flat, bb_flat, attrs, mn)
    out1 = out1.reshape(KEEP_PAD, 16)

    dets = jnp.concatenate([out2[:MAX_NUM, 0:4], out1[:MAX_NUM, 4:5]], axis=1)
    labels = out1[:MAX_NUM, 5].astype(jnp.int32)
    depths = out2[:MAX_NUM, 6:7]
    dims = out2[:MAX_NUM, 7:10]
    rots = out2[:MAX_NUM, 10:11]
    cen_2ds = out2[:MAX_NUM, 11:13]
    return (dets, labels, depths, dims, rots, cen_2ds)


# speedup vs baseline: 1.1665x; 1.1665x over previous
import functools

import jax
import jax.numpy as jnp
import numpy as np
from jax import lax
from jax.experimental import pallas as pl
from jax.experimental.pallas import tpu as pltpu
from jax.experimental.pallas import tpu_sc as plsc

N_PROP = 5000
N_CLS = 8
N_CAND = N_PROP * N_CLS
NT = 16
CHUNK = 2560
ROWS_PER_TILE = CHUNK // N_CLS
NV = CHUNK // 16
SCORE_THR = 0.95
IOU_THR = 0.5
MAX_NUM = 100
KEEP_PAD = 112
N_ATTR = 16
NREC = 8
BIG = np.int32(np.iinfo(np.int32).max)
NEG_INF = np.float32(-np.inf)
F0 = np.float32(0.0)


def _sc_body(scores_hbm, bb_hbm, attrs_hbm, mn_hbm,
             out1_hbm, out2_hbm,
             sch9, bbch, scorec, candrec,
             recv, allv, allv2, keepi, keeps, rowidx, attrrows, outv, mnv,
             bufA, winners, sem):
    sid = lax.axis_index("s")
    base = sid * CHUNK
    lane = jnp.arange(16, dtype=jnp.int32)
    zeros16i = jnp.zeros((16,), jnp.int32)
    ones16i = jnp.full((16,), 1, jnp.int32)

    pltpu.sync_copy(
        scores_hbm.at[pl.ds(sid * ROWS_PER_TILE * 9, ROWS_PER_TILE * 9)], sch9)
    pltpu.sync_copy(
        bb_hbm.at[pl.ds(sid * ROWS_PER_TILE * 4, ROWS_PER_TILE * 4)], bbch)
    pltpu.sync_copy(mn_hbm, mnv)

    def initb(k, _):
        scorec[pl.ds(k * 16, 16)] = jnp.full((16,), NEG_INF, jnp.float32)
        return 0
    lax.fori_loop(0, NV, initb, 0)

    def scan_body(k, carry):
        off, mc, ffl = carry
        j = k * 16 + lane
        rl = j >> 3
        cls = j & 7
        v = plsc.load_gather(sch9, [rl * 9 + 1 + cls])
        msk = v > SCORE_THR
        fidx_v = base + j
        cum = plsc.cumsum(msk.astype(jnp.int32))
        pos = off + cum - 1
        plsc.store_scatter(candrec, [pos * NREC],
                           plsc.bitcast(fidx_v, jnp.float32), mask=msk)
        plsc.store_scatter(scorec, [pos], v, mask=msk)
        cnt = plsc.all_reduce_population_count(msk)
        bx1 = plsc.load_gather(bbch, [rl * 4])
        by1 = plsc.load_gather(bbch, [rl * 4 + 1])
        bx2 = plsc.load_gather(bbch, [rl * 4 + 2])
        by2 = plsc.load_gather(bbch, [rl * 4 + 3])
        cmax = jnp.maximum(jnp.maximum(bx1, by1), jnp.maximum(bx2, by2))
        mc = jnp.maximum(mc, jnp.where(msk, cmax, NEG_INF))
        ffl = jnp.minimum(ffl, jnp.where(msk, fidx_v, BIG))
        return (off + cnt, mc, ffl)

    off_v, mc_v, ffl_v = lax.fori_loop(
        0, NV, scan_body,
        (zeros16i, jnp.full((16,), NEG_INF, jnp.float32),
         jnp.full((16,), BIG, jnp.int32)))
    nloc = jnp.max(off_v)
    mcl = jnp.max(mc_v)
    ffl = jnp.min(ffl_v)
    fsl = jnp.max(plsc.load_gather(scorec, [zeros16i]))
    fsl = jnp.where(nloc > 0, fsl, F0)
    s0 = jnp.max(plsc.load_gather(sch9, [ones16i]))

    rec = jnp.full((16,), F0, jnp.float32)
    rec = jnp.where(lane == 0, mcl, rec)
    rec = jnp.where(lane == 1,
                    plsc.bitcast(jnp.full((16,), ffl, jnp.int32), jnp.float32),
                    rec)
    rec = jnp.where(lane == 2, fsl, rec)
    rec = jnp.where(lane == 3, s0, rec)
    recv[pl.ds(0, 16)] = rec
    pltpu.sync_copy(recv.at[pl.ds(0, 16)], bufA.at[pl.ds(sid * 16, 16)])
    plsc.subcore_barrier()
    pltpu.sync_copy(bufA, allv)

    def field_f(f):
        return plsc.load_gather(allv, [lane * 16 + f])

    mc_g = jnp.max(field_f(0))
    ffl_all = plsc.bitcast(field_f(1), jnp.int32)
    gfill = jnp.min(ffl_all)
    gfs = jnp.max(jnp.where(ffl_all == gfill, field_f(2), NEG_INF))
    s0g = jnp.max(jnp.where(lane == 0, field_f(3), NEG_INF))
    none_valid = gfill == BIG
    gf = jnp.where(none_valid, 0, gfill)
    gfs = jnp.where(none_valid, s0g, gfs)
    offc = mc_g + np.float32(1.0)

    nvl = (nloc + 15) // 16

    def gco(k, _):
        cid = k * 16 + lane
        idxv = plsc.bitcast(plsc.load_gather(candrec, [cid * NREC]), jnp.int32)
        lidx = jnp.minimum(jnp.maximum(idxv - base, 0), CHUNK - 1)
        rl = lidx >> 3
        labf = (idxv & 7).astype(jnp.float32)
        offv = labf * offc
        a = plsc.load_gather(bbch, [rl * 4]) + offv
        b = plsc.load_gather(bbch, [rl * 4 + 1]) + offv
        c = plsc.load_gather(bbch, [rl * 4 + 2]) + offv
        d = plsc.load_gather(bbch, [rl * 4 + 3]) + offv
        area = jnp.maximum(c - a, F0) * jnp.maximum(d - b, F0)
        plsc.store_scatter(candrec, [cid * NREC + 1], a)
        plsc.store_scatter(candrec, [cid * NREC + 2], b)
        plsc.store_scatter(candrec, [cid * NREC + 3], c)
        plsc.store_scatter(candrec, [cid * NREC + 4], d)
        plsc.store_scatter(candrec, [cid * NREC + 5], area)
        return 0
    lax.fori_loop(0, nvl, gco, 0)

    def pfill(k, _):
        keepi[pl.ds(k * 16, 16)] = jnp.full((16,), gf, jnp.int32)
        keeps[pl.ds(k * 16, 16)] = jnp.full((16,), gfs, jnp.float32)
        return 0
    lax.fori_loop(0, KEEP_PAD, pfill, 0)

    mn_s = jnp.max(mnv[...])
    limit = jnp.minimum(jnp.int32(MAX_NUM), mn_s)

    recmap = jnp.minimum(jnp.maximum(lane - 1, 0), NREC - 1)

    def cond(carry):
        t, e, alive = carry
        return jnp.logical_and(t < limit, alive == 1)

    def body(carry):
        t, e, _ = carry

        def am(k, c2):
            best, bk = c2
            v = scorec[pl.ds(k * 16, 16)]
            upd = v > best
            return (jnp.where(upd, v, best),
                    jnp.where(upd, jnp.full((16,), k, jnp.int32), bk))

        def top1(excl_pos):
            def am2(k, c2):
                best, bk = c2
                v = scorec[pl.ds(k * 16, 16)]
                v = jnp.where(k * 16 + lane == excl_pos, NEG_INF, v)
                upd = v > best
                return (jnp.where(upd, v, best),
                        jnp.where(upd, jnp.full((16,), k, jnp.int32), bk))
            best, bk = lax.fori_loop(
                0, nvl, am2,
                (jnp.full((16,), NEG_INF, jnp.float32), zeros16i))
            m_l = jnp.max(best)
            pos_pl = bk * 16 + lane
            pfv = plsc.bitcast(plsc.load_gather(candrec, [pos_pl * NREC]),
                               jnp.int32)
            tied = best == m_l
            jl = jnp.min(jnp.where(tied, pfv, BIG))
            posl = jnp.min(
                jnp.where(jnp.logical_and(tied, pfv == jl), pos_pl, BIG))
            posc = jnp.minimum(jnp.maximum(posl, 0), CHUNK - 1)
            r = plsc.load_gather(
                candrec, [jnp.full((16,), posc * NREC, jnp.int32) + recmap])
            return jnp.where(lane == 0, m_l, r), posc, m_l

        r1, pos1, m1 = top1(jnp.int32(-1))
        pos1x = jnp.where(m1 > NEG_INF, pos1, -1)
        r2, _, _ = top1(pos1x)
        recv[pl.ds(0, 16)] = r1
        recv[pl.ds(16, 16)] = r2
        eb = jnp.bitwise_and(e, 1)
        pltpu.sync_copy(recv, winners.at[pl.ds(eb * 512 + sid * 32, 32)])
        plsc.subcore_barrier()
        pltpu.sync_copy(winners.at[pl.ds(eb * 512, 512)], allv2)

        def f2(off):
            return plsc.load_gather(allv2, [lane * 32 + off])

        sco1 = f2(0)
        idx1 = plsc.bitcast(f2(1), jnp.int32)
        m_g = jnp.max(sco1)
        alive = m_g > NEG_INF
        gtied = sco1 == m_g
        jwin = jnp.min(jnp.where(gtied, idx1, BIG))
        sel = jnp.logical_and(gtied, idx1 == jwin)
        w32 = jnp.minimum(jnp.min(jnp.where(sel, lane, 16)), 15) * 32

        def wf(off):
            return plsc.load_gather(allv2,
                                    [jnp.full((16,), w32 + off, jnp.int32)])

        bx1, by1, bx2, by2, bar = wf(2), wf(3), wf(4), wf(5), wf(6)

        x1a, y1a, x2a, y2a, ara = f2(2), f2(3), f2(4), f2(5), f2(6)
        sco2 = f2(16)
        idx2 = plsc.bitcast(f2(17), jnp.int32)
        x1b, y1b, x2b, y2b, arb = f2(18), f2(19), f2(20), f2(21), f2(22)

        def iou_vs_w1(xa, ya, xb, yb, ar):
            xx1 = jnp.maximum(bx1, xa)
            yy1 = jnp.maximum(by1, ya)
            xx2 = jnp.minimum(bx2, xb)
            yy2 = jnp.minimum(by2, yb)
            inter = (jnp.maximum(xx2 - xx1, F0) *
                     jnp.maximum(yy2 - yy1, F0))
            union = bar + ar - inter
            return inter / jnp.maximum(union, np.float32(1e-6))

        iouA = iou_vs_w1(x1a, y1a, x2a, y2a, ara)
        iouB = iou_vs_w1(x1b, y1b, x2b, y2b, arb)
        effA = jnp.where(jnp.logical_or(iouA > IOU_THR, idx1 == jwin),
                         NEG_INF, sco1)
        effB = jnp.where(jnp.logical_or(iouB > IOU_THR, idx2 == jwin),
                         NEG_INF, sco2)
        blocked = jnp.logical_and(
            jnp.logical_and(effA == NEG_INF, effB == NEG_INF),
            sco2 > NEG_INF)
        anyblk = jnp.max(jnp.where(blocked, 1, 0))
        m2 = jnp.max(jnp.maximum(effA, effB))
        take2 = jnp.logical_and(
            jnp.logical_and(alive, m2 > NEG_INF),
            jnp.logical_and(anyblk == 0, t + 1 < limit))
        j2win = jnp.min(jnp.minimum(jnp.where(effA == m2, idx1, BIG),
                                    jnp.where(effB == m2, idx2, BIG)))
        r2pos = jnp.min(jnp.minimum(
            jnp.where(jnp.logical_and(effA == m2, idx1 == j2win),
                      lane * 32, BIG),
            jnp.where(jnp.logical_and(effB == m2, idx2 == j2win),
                      lane * 32 + 16, BIG)))
        r2base = jnp.minimum(jnp.maximum(r2pos, 0), 511 - 8)

        def w2f(off):
            return plsc.load_gather(allv2,
                                    [jnp.full((16,), r2base + off, jnp.int32)])

        cx1, cy1, cx2, cy2, car = w2f(2), w2f(3), w2f(4), w2f(5), w2f(6)

        @pl.when(alive)
        def _():
            keepi[pl.ds(t * 16, 16)] = jnp.full((16,), jwin, jnp.int32)
            keeps[pl.ds(t * 16, 16)] = jnp.full((16,), m_g, jnp.float32)

            @pl.when(take2)
            def _():
                keepi[pl.ds((t + 1) * 16, 16)] = jnp.full((16,), j2win,
                                                          jnp.int32)
                keeps[pl.ds((t + 1) * 16, 16)] = jnp.full((16,), m2,
                                                          jnp.float32)

            def sup(k, _):
                c8 = (k * 16 + lane) * NREC
                fi = plsc.bitcast(plsc.load_gather(candrec, [c8]), jnp.int32)
                a1 = plsc.load_gather(candrec, [c8 + 1])
                b1 = plsc.load_gather(candrec, [c8 + 2])
                a2 = plsc.load_gather(candrec, [c8 + 3])
                b2 = plsc.load_gather(candrec, [c8 + 4])
                ar = plsc.load_gather(candrec, [c8 + 5])
                sc = scorec[pl.ds(k * 16, 16)]
                xx1 = jnp.maximum(bx1, a1)
                yy1 = jnp.maximum(by1, b1)
                xx2 = jnp.minimum(bx2, a2)
                yy2 = jnp.minimum(by2, b2)
                inter = (jnp.maximum(xx2 - xx1, F0) *
                         jnp.maximum(yy2 - yy1, F0))
                union = bar + ar - inter
                iou = inter / jnp.maximum(union, np.float32(1e-6))
                kill = jnp.logical_or(iou > IOU_THR, fi == jwin)
                ux1 = jnp.maximum(cx1, a1)
                uy1 = jnp.maximum(cy1, b1)
                ux2 = jnp.minimum(cx2, a2)
                uy2 = jnp.minimum(cy2, b2)
                uin = (jnp.maximum(ux2 - ux1, F0) *
                       jnp.maximum(uy2 - uy1, F0))
                uun = car + ar - uin
                iou2 = uin / jnp.maximum(uun, np.float32(1e-6))
                kill2 = jnp.logical_and(
                    take2, jnp.logical_or(iou2 > IOU_THR, fi == j2win))
                kill = jnp.logical_or(kill, kill2)
                scorec[pl.ds(k * 16, 16)] = jnp.where(kill, NEG_INF, sc)
                return 0
            lax.fori_loop(0, nvl, sup, 0)

        tnew = t + 1 + jnp.where(jnp.logical_and(alive, take2), 1, 0)
        return (tnew, e + 1, alive.astype(jnp.int32))

    lax.while_loop(cond, body, (jnp.int32(0), jnp.int32(0), jnp.int32(1)))

    @pl.when(sid == 0)
    def _():
        def ri(k, _):
            slots = k * 16 + lane
            kvv = plsc.load_gather(keepi, [slots * 16])
            rowidx[pl.ds(k * 16, 16)] = kvv >> 3
            return 0
        lax.fori_loop(0, KEEP_PAD // 16, ri, 0)
        pltpu.async_copy(attrs_hbm.at[rowidx], attrrows, sem).wait()

        def emit(tt, _):
            kv = keepi[pl.ds(tt * 16, 16)]
            sv = keeps[pl.ds(tt * 16, 16)]
            labf = (kv & 7).astype(jnp.float32)
            row = jnp.where(lane == 4, sv,
                            jnp.where(lane == 5, labf,
                                      jnp.zeros((16,), jnp.float32)))
            outv[pl.ds(tt * 16, 16)] = row
            return 0
        lax.fori_loop(0, KEEP_PAD, emit, 0)
        pltpu.sync_copy(outv, out1_hbm)
        pltpu.sync_copy(attrrows, out2_hbm)


_mesh = plsc.VectorSubcoreMesh(core_axis_name="c", subcore_axis_name="s")

_sc_call = functools.partial(
    pl.kernel,
    mesh=_mesh,
    out_type=[
        jax.ShapeDtypeStruct((KEEP_PAD * 16,), jnp.float32),
        jax.ShapeDtypeStruct((KEEP_PAD, N_ATTR), jnp.float32),
    ],
    compiler_params=pltpu.CompilerParams(needs_layout_passes=False,
                                         use_tc_tiling_on_sc=False),
    scratch_types=[
        pltpu.VMEM((ROWS_PER_TILE * 9,), jnp.float32),
        pltpu.VMEM((ROWS_PER_TILE * 4,), jnp.float32),
        pltpu.VMEM((CHUNK,), jnp.float32),
        pltpu.VMEM((CHUNK * NREC,), jnp.float32),
        pltpu.VMEM((32,), jnp.float32),
        pltpu.VMEM((256,), jnp.float32),
        pltpu.VMEM((512,), jnp.float32),
        pltpu.VMEM((KEEP_PAD * 16,), jnp.int32),
        pltpu.VMEM((KEEP_PAD * 16,), jnp.float32),
        pltpu.VMEM((KEEP_PAD,), jnp.int32),
        pltpu.VMEM((KEEP_PAD, N_ATTR), jnp.float32),
        pltpu.VMEM((KEEP_PAD * 16,), jnp.float32),
        pltpu.VMEM((16,), jnp.int32),
        pltpu.VMEM_SHARED((256,), jnp.float32),
        pltpu.VMEM_SHARED((2048,), jnp.float32),
        pltpu.SemaphoreType.DMA,
    ],
)(_sc_body)


@jax.jit
def kernel(multi_bboxes, multi_scores, depth_pred, dim_pred, rot_pred,
           cen_2d_pred, max_num):
    attrs = jnp.concatenate([
        multi_bboxes,
        jnp.zeros((N_PROP, 2), jnp.float32),
        depth_pred,
        dim_pred,
        rot_pred,
        cen_2d_pred,
        jnp.zeros((N_PROP, 3), jnp.float32),
    ], axis=1)

    mn = jnp.full((16,), jnp.asarray(max_num, jnp.int32))

    n_pad_rows = NT * ROWS_PER_TILE - N_PROP
    s_flat = jnp.pad(multi_scores.reshape(-1), (0, n_pad_rows * 9))
    bb_flat = jnp.pad(multi_bboxes.reshape(-1), (0, n_pad_rows * 4))
    out1, out2 = _sc_call(s_flat, bb_flat, attrs, mn)
    out1 = out1.reshape(KEEP_PAD, 16)

    dets = jnp.concatenate([out2[:MAX_NUM, 0:4], out1[:MAX_NUM, 4:5]], axis=1)
    labels = out1[:MAX_NUM, 5].astype(jnp.int32)
    depths = out2[:MAX_NUM, 6:7]
    dims = out2[:MAX_NUM, 7:10]
    rots = out2[:MAX_NUM, 10:11]
    cen_2ds = out2[:MAX_NUM, 11:13]
    return (dets, labels, depths, dims, rots, cen_2ds)
